# Initial kernel scaffold; baseline (speedup 1.0000x reference)
import jax
import jax.numpy as jnp
from jax.experimental import pallas as pl


def kernel(x, gate_w, dense_1_w, dense_1_b, dense_2_w, dense_2_b):
    raise NotImplementedError("write your pallas kernel here")


# speedup vs baseline: 5.7927x; 5.7927x over previous
import jax
import jax.numpy as jnp
from jax.experimental import pallas as pl
from jax.experimental.pallas import tpu as pltpu

_NE = 8
_D = 768
_F = 1024
_T = 64
_C = 256
_J = _F // _C
_ALPHA = 1.702
_LIMIT = 7.0


def _body(x_ref, gw_ref, w1_ref, b1g_ref, b1u_ref, w2_ref, b2_ref,
          out_ref, s_ref):
    e = pl.program_id(0)
    j = pl.program_id(1)

    @pl.when((e == 0) & (j == 0))
    def _init():
        gate = jnp.dot(x_ref[...], gw_ref[...], preferred_element_type=jnp.float32)
        idx = jax.lax.broadcasted_iota(jnp.int32, (_T, _NE), 1)
        v1 = jnp.max(gate, axis=1, keepdims=True)
        i1 = jnp.min(jnp.where(gate == v1, idx, _NE), axis=1, keepdims=True)
        masked = jnp.where(idx == i1, -jnp.inf, gate)
        v2 = jnp.max(masked, axis=1, keepdims=True)
        i2 = jnp.min(jnp.where(masked == v2, idx, _NE), axis=1, keepdims=True)
        t = jnp.exp(v2 - v1)
        den = 1.0 + t
        s_ref[...] = (jnp.where(idx == i1, 1.0, 0.0)
                      + jnp.where(idx == i2, t, 0.0)) / den
        out_ref[...] = jnp.zeros_like(out_ref)

    x = x_ref[...]
    w1 = w1_ref[0]
    g = jax.lax.dot_general(x, w1[:, :_D], (((1,), (1,)), ((), ())),
                            preferred_element_type=jnp.float32)
    u = jax.lax.dot_general(x, w1[:, _D:], (((1,), (1,)), ((), ())),
                            preferred_element_type=jnp.float32)
    g = jnp.minimum(g + b1g_ref[0], _LIMIT)
    u = jnp.clip(u + b1u_ref[0], -_LIMIT, _LIMIT)
    act = g * (1.0 / (1.0 + jnp.exp(-_ALPHA * g))) * (u + 1.0)
    part = jax.lax.dot_general(act, w2_ref[0], (((1,), (1,)), ((), ())),
                               preferred_element_type=jnp.float32)
    idx = jax.lax.broadcasted_iota(jnp.int32, (_T, _NE), 1)
    s_col = jnp.sum(jnp.where(idx == e, s_ref[...], 0.0), axis=1, keepdims=True)
    part = part + (j == 0).astype(jnp.float32) * b2_ref[0]
    out_ref[...] += s_col * part


def kernel(x, gate_w, dense_1_w, dense_1_b, dense_2_w, dense_2_b):
    B, L, D = x.shape
    x_f = x.reshape(B * L, D)
    w1r = dense_1_w.reshape(_NE, _F, 2 * _D)
    b1g = dense_1_b[:, 0::2].reshape(_NE * _J, 1, _C)
    b1u = dense_1_b[:, 1::2].reshape(_NE * _J, 1, _C)
    b2r = dense_2_b.reshape(_NE, 1, _D)
    out = pl.pallas_call(
        _body,
        grid=(_NE, _J),
        in_specs=[
            pl.BlockSpec((_T, _D), lambda e, j: (0, 0)),
            pl.BlockSpec((_D, _NE), lambda e, j: (0, 0)),
            pl.BlockSpec((1, _C, 2 * _D), lambda e, j: (e, j, 0)),
            pl.BlockSpec((1, 1, _C), lambda e, j: (e * _J + j, 0, 0)),
            pl.BlockSpec((1, 1, _C), lambda e, j: (e * _J + j, 0, 0)),
            pl.BlockSpec((1, _D, _C), lambda e, j: (e, 0, j)),
            pl.BlockSpec((1, 1, _D), lambda e, j: (e, 0, 0)),
        ],
        out_specs=pl.BlockSpec((_T, _D), lambda e, j: (0, 0)),
        out_shape=jax.ShapeDtypeStruct((_T, _D), jnp.float32),
        scratch_shapes=[pltpu.VMEM((_T, _NE), jnp.float32)],
        compiler_params=pltpu.CompilerParams(
            dimension_semantics=("arbitrary", "arbitrary")),
    )(x_f, gate_w, w1r, b1g, b1u, dense_2_w, b2r)
    return out.reshape(B, L, D)


# speedup vs baseline: 6.3604x; 1.0980x over previous
import jax
import jax.numpy as jnp
from jax.experimental import pallas as pl
from jax.experimental.pallas import tpu as pltpu

_NE = 8
_D = 768
_F = 1024
_T = 64
_C = 512
_J = _F // _C
_ALPHA = 1.702
_LIMIT = 7.0


def _body(x_ref, gw_ref, w1_ref, b1g_ref, b1u_ref, w2_ref, b2_ref,
          out_ref, s_ref):
    e = pl.program_id(0)
    j = pl.program_id(1)

    @pl.when((e == 0) & (j == 0))
    def _init():
        gate = jnp.dot(x_ref[...], gw_ref[...], preferred_element_type=jnp.float32)
        idx = jax.lax.broadcasted_iota(jnp.int32, (_T, _NE), 1)
        v1 = jnp.max(gate, axis=1, keepdims=True)
        i1 = jnp.min(jnp.where(gate == v1, idx, _NE), axis=1, keepdims=True)
        masked = jnp.where(idx == i1, -jnp.inf, gate)
        v2 = jnp.max(masked, axis=1, keepdims=True)
        i2 = jnp.min(jnp.where(masked == v2, idx, _NE), axis=1, keepdims=True)
        t = jnp.exp(v2 - v1)
        den = 1.0 + t
        s_ref[...] = (jnp.where(idx == i1, 1.0, 0.0)
                      + jnp.where(idx == i2, t, 0.0)) / den
        out_ref[...] = jnp.zeros_like(out_ref)

    x = x_ref[...]
    w1 = w1_ref[0]
    g = jax.lax.dot_general(x, w1[:, :_D], (((1,), (1,)), ((), ())),
                            preferred_element_type=jnp.float32)
    u = jax.lax.dot_general(x, w1[:, _D:], (((1,), (1,)), ((), ())),
                            preferred_element_type=jnp.float32)
    g = jnp.minimum(g + b1g_ref[0], _LIMIT)
    u = jnp.clip(u + b1u_ref[0], -_LIMIT, _LIMIT)
    act = g * (1.0 / (1.0 + jnp.exp(-_ALPHA * g))) * (u + 1.0)
    part = jax.lax.dot_general(act, w2_ref[0], (((1,), (1,)), ((), ())),
                               preferred_element_type=jnp.float32)
    idx = jax.lax.broadcasted_iota(jnp.int32, (_T, _NE), 1)
    s_col = jnp.sum(jnp.where(idx == e, s_ref[...], 0.0), axis=1, keepdims=True)
    part = part + (j == 0).astype(jnp.float32) * b2_ref[0]
    out_ref[...] += s_col * part


def kernel(x, gate_w, dense_1_w, dense_1_b, dense_2_w, dense_2_b):
    B, L, D = x.shape
    x_f = x.reshape(B * L, D)
    w1r = dense_1_w.reshape(_NE, _F, 2 * _D)
    b1g = dense_1_b[:, 0::2].reshape(_NE * _J, 1, _C)
    b1u = dense_1_b[:, 1::2].reshape(_NE * _J, 1, _C)
    b2r = dense_2_b.reshape(_NE, 1, _D)
    out = pl.pallas_call(
        _body,
        grid=(_NE, _J),
        in_specs=[
            pl.BlockSpec((_T, _D), lambda e, j: (0, 0)),
            pl.BlockSpec((_D, _NE), lambda e, j: (0, 0)),
            pl.BlockSpec((1, _C, 2 * _D), lambda e, j: (e, j, 0)),
            pl.BlockSpec((1, 1, _C), lambda e, j: (e * _J + j, 0, 0)),
            pl.BlockSpec((1, 1, _C), lambda e, j: (e * _J + j, 0, 0)),
            pl.BlockSpec((1, _D, _C), lambda e, j: (e, 0, j)),
            pl.BlockSpec((1, 1, _D), lambda e, j: (e, 0, 0)),
        ],
        out_specs=pl.BlockSpec((_T, _D), lambda e, j: (0, 0)),
        out_shape=jax.ShapeDtypeStruct((_T, _D), jnp.float32),
        scratch_shapes=[pltpu.VMEM((_T, _NE), jnp.float32)],
        compiler_params=pltpu.CompilerParams(
            dimension_semantics=("arbitrary", "arbitrary")),
    )(x_f, gate_w, w1r, b1g, b1u, dense_2_w, b2r)
    return out.reshape(B, L, D)


# speedup vs baseline: 6.5294x; 1.0266x over previous
import jax
import jax.numpy as jnp
from jax.experimental import pallas as pl
from jax.experimental.pallas import tpu as pltpu

_NE = 8
_D = 768
_F = 1024
_T = 64
_C = 1024
_J = _F // _C
_ALPHA = 1.702
_LIMIT = 7.0


def _body(x_ref, gw_ref, w1_ref, b1g_ref, b1u_ref, w2_ref, b2_ref,
          out_ref, s_ref):
    e = pl.program_id(0)
    j = pl.program_id(1)

    @pl.when((e == 0) & (j == 0))
    def _init():
        gate = jnp.dot(x_ref[...], gw_ref[...], preferred_element_type=jnp.float32)
        idx = jax.lax.broadcasted_iota(jnp.int32, (_T, _NE), 1)
        v1 = jnp.max(gate, axis=1, keepdims=True)
        i1 = jnp.min(jnp.where(gate == v1, idx, _NE), axis=1, keepdims=True)
        masked = jnp.where(idx == i1, -jnp.inf, gate)
        v2 = jnp.max(masked, axis=1, keepdims=True)
        i2 = jnp.min(jnp.where(masked == v2, idx, _NE), axis=1, keepdims=True)
        t = jnp.exp(v2 - v1)
        den = 1.0 + t
        s_ref[...] = (jnp.where(idx == i1, 1.0, 0.0)
                      + jnp.where(idx == i2, t, 0.0)) / den
        out_ref[...] = jnp.zeros_like(out_ref)

    x = x_ref[...]
    w1 = w1_ref[0]
    g = jax.lax.dot_general(x, w1[:, :_D], (((1,), (1,)), ((), ())),
                            preferred_element_type=jnp.float32)
    u = jax.lax.dot_general(x, w1[:, _D:], (((1,), (1,)), ((), ())),
                            preferred_element_type=jnp.float32)
    g = jnp.minimum(g + b1g_ref[0], _LIMIT)
    u = jnp.clip(u + b1u_ref[0], -_LIMIT, _LIMIT)
    act = g * (1.0 / (1.0 + jnp.exp(-_ALPHA * g))) * (u + 1.0)
    part = jax.lax.dot_general(act, w2_ref[0], (((1,), (1,)), ((), ())),
                               preferred_element_type=jnp.float32)
    idx = jax.lax.broadcasted_iota(jnp.int32, (_T, _NE), 1)
    s_col = jnp.sum(jnp.where(idx == e, s_ref[...], 0.0), axis=1, keepdims=True)
    part = part + (j == 0).astype(jnp.float32) * b2_ref[0]
    out_ref[...] += s_col * part


def kernel(x, gate_w, dense_1_w, dense_1_b, dense_2_w, dense_2_b):
    B, L, D = x.shape
    x_f = x.reshape(B * L, D)
    w1r = dense_1_w.reshape(_NE, _F, 2 * _D)
    b1g = dense_1_b[:, 0::2].reshape(_NE * _J, 1, _C)
    b1u = dense_1_b[:, 1::2].reshape(_NE * _J, 1, _C)
    b2r = dense_2_b.reshape(_NE, 1, _D)
    out = pl.pallas_call(
        _body,
        grid=(_NE, _J),
        in_specs=[
            pl.BlockSpec((_T, _D), lambda e, j: (0, 0)),
            pl.BlockSpec((_D, _NE), lambda e, j: (0, 0)),
            pl.BlockSpec((1, _C, 2 * _D), lambda e, j: (e, j, 0)),
            pl.BlockSpec((1, 1, _C), lambda e, j: (e * _J + j, 0, 0)),
            pl.BlockSpec((1, 1, _C), lambda e, j: (e * _J + j, 0, 0)),
            pl.BlockSpec((1, _D, _C), lambda e, j: (e, 0, j)),
            pl.BlockSpec((1, 1, _D), lambda e, j: (e, 0, 0)),
        ],
        out_specs=pl.BlockSpec((_T, _D), lambda e, j: (0, 0)),
        out_shape=jax.ShapeDtypeStruct((_T, _D), jnp.float32),
        scratch_shapes=[pltpu.VMEM((_T, _NE), jnp.float32)],
        compiler_params=pltpu.CompilerParams(
            dimension_semantics=("arbitrary", "arbitrary")),
    )(x_f, gate_w, w1r, b1g, b1u, dense_2_w, b2r)
    return out.reshape(B, L, D)
